# baseline (speedup 1.0000x reference)
import functools

import jax
import jax.numpy as jnp
from jax import lax
from jax.experimental import pallas as pl
from jax.experimental.pallas import tpu as pltpu
from jax.experimental.pallas import tpu_sc as plsc

D = 128
L = 16
CHUNK = 128
NB = 5
NC = 2
NS = 16
NW = NC * NS
RPT = 312
PAD_E = 2048
SLAB = 336
DUMMY = 329

PERM = []
for _k in range(D // (2 * L)):
    PERM += [2 * L * _k + 2 * _i for _i in range(L)]
    PERM += [2 * L * _k + 2 * _i + 1 for _i in range(L)]


def _count_body(nblk, er_ref, out_ref):
    ids = er_ref[...]
    for w in range(NW + 1):
        bound = RPT * w if w < NW else 10000
        cnt = jnp.sum((ids < bound).astype(jnp.int32))
        out_ref[w] = jnp.full((D,), cnt, jnp.int32)
    for w in range(NW + 1, 40):
        out_ref[w] = jnp.zeros((D,), jnp.int32)


def _count(edge_row_p):
    EP = edge_row_p.shape[0]
    nblk = EP // D
    return pl.pallas_call(
        functools.partial(_count_body, nblk),
        in_specs=[pl.BlockSpec((nblk, D), lambda: (0, 0))],
        out_specs=pl.BlockSpec((40, D), lambda: (0, 0)),
        out_shape=jax.ShapeDtypeStruct((40, D), jnp.int32),
    )(edge_row_p.reshape(nblk, D))


def _spmm_body(N, emb_hbm, vals_hbm, row_hbm, col_hbm, bnd_hbm, out_hbm,
               rows, cols, rowid, vals, bndv, slab, gsem, isem):
    c = lax.axis_index("c")
    s = lax.axis_index("s")
    w = s * NC + c
    rowbase = w * RPT
    nrows = jnp.where(w == NW - 1, RPT + 16, RPT)

    pltpu.sync_copy(bnd_hbm, bndv)
    estart = bndv[w, pl.ds(0, L)][0]
    eend = bndv[w + 1, pl.ds(0, L)][0]
    ea = (estart // 8) * 8
    nch = jnp.maximum(lax.div(eend - ea + CHUNK - 1, CHUNK), 1)
    nloop = lax.div(nch + NB - 1, NB)

    def _zero_row(i, carry):
        for k in range(D // L):
            slab[i, pl.ds(k * L, L)] = jnp.zeros((L,), jnp.float32)
        return carry
    lax.fori_loop(0, SLAB, _zero_row, 0)

    def _idx_copies(b, jj):
        base = ea + jj * CHUNK
        return (
            (col_hbm.at[pl.ds(base, CHUNK)], cols[b]),
            (row_hbm.at[pl.ds(base, CHUNK)], rowid[b]),
            (vals_hbm.at[pl.ds(base, CHUNK)], vals[b]),
        )

    def _start_idx(b, jj):
        for src, dst in _idx_copies(b, jj):
            pltpu.async_copy(src, dst, isem[b])

    def _wait_idx(b, jj):
        for src, dst in _idx_copies(b, jj):
            pltpu.make_async_copy(src, dst, isem[b]).wait()

    def _start_gather(b):
        pltpu.async_copy(emb_hbm.at[cols[b]], rows[b], gsem[b])

    def _wait_gather(b):
        pltpu.make_async_copy(emb_hbm.at[cols[b]], rows[b], gsem[b]).wait()

    def _process(b):
        def _grp(g, carry):
            acc = [jnp.zeros((L,), jnp.float32) for _ in range(D // L)]
            cur = jnp.int32(DUMMY)
            vgrp = vals[b][pl.ds(g * L, L)]
            rgrp = rowid[b][pl.ds(g * L, L)]
            for e16 in range(L):
                row_e = rgrp[e16]
                sval = vgrp[e16]
                rel = row_e - rowbase
                oor = (rel < 0) | (rel >= nrows)
                rel_c = jnp.where(oor, DUMMY, rel)
                sval = jnp.where(oor, 0.0, sval)
                pred = rel_c != cur
                if e16:
                    @pl.when(pred)
                    def _flush():
                        for k in range(D // L):
                            sl = pl.ds(k * L, L)
                            slab[cur, sl] = slab[cur, sl] + acc[k]
                e = g * L + e16
                keep = jnp.where(pred, 0.0, 1.0)
                for k4 in range(D // (2 * L)):
                    xi = rows[b][e, pl.ds(k4 * L, L)]
                    lo = lax.bitcast_convert_type(xi << 16, jnp.float32)
                    hi = lax.bitcast_convert_type(
                        xi & jnp.int32(-65536), jnp.float32)
                    acc[2 * k4] = lo * sval + acc[2 * k4] * keep
                    acc[2 * k4 + 1] = hi * sval + acc[2 * k4 + 1] * keep
                cur = jnp.where(pred, rel_c, cur)
            for k in range(D // L):
                sl = pl.ds(k * L, L)
                slab[cur, sl] = slab[cur, sl] + acc[k]
            return carry
        lax.fori_loop(0, CHUNK // L, _grp, 0)

    for m in range(NB - 1):
        _start_idx(m, m)
    for m in range(NB - 2):
        _wait_idx(m, m)
        _start_gather(m)

    def _ring(j, carry):
        for b in range(NB):
            jj = j * NB + b
            _wait_gather(b)
            _start_idx((b + NB - 1) % NB, jj + NB - 1)
            _wait_idx((b + NB - 2) % NB, jj + NB - 2)
            _start_gather((b + NB - 2) % NB)
            _process(b)
        return carry
    lax.fori_loop(0, nloop, _ring, 0)

    for m in range(NB - 2):
        _wait_gather(m)
    _wait_idx(NB - 2, nloop * NB + NB - 2)

    pltpu.sync_copy(slab.at[pl.ds(0, RPT)], out_hbm.at[pl.ds(rowbase, RPT)])
    @pl.when(w == NW - 1)
    def _write_tail():
        pltpu.sync_copy(slab.at[pl.ds(RPT, 16)],
                        out_hbm.at[pl.ds(rowbase + RPT, 16)])


def _spmm(emb, vals_p, row_p, col_p, bnd):
    N = emb.shape[0]
    mesh = plsc.VectorSubcoreMesh(core_axis_name="c", subcore_axis_name="s")
    f = pl.kernel(
        functools.partial(_spmm_body, N),
        out_type=jax.ShapeDtypeStruct((N, D), jnp.float32),
        mesh=mesh,
        compiler_params=pltpu.CompilerParams(use_tc_tiling_on_sc=False),
        scratch_types=[
            [pltpu.VMEM((CHUNK, D // 2), jnp.int32) for _ in range(NB)],
            [pltpu.VMEM((CHUNK,), jnp.int32) for _ in range(NB)],
            [pltpu.VMEM((CHUNK,), jnp.int32) for _ in range(NB)],
            [pltpu.VMEM((CHUNK,), jnp.float32) for _ in range(NB)],
            pltpu.VMEM((40, D), jnp.int32),
            pltpu.VMEM((SLAB, D), jnp.float32),
            [pltpu.SemaphoreType.DMA for _ in range(NB)],
            [pltpu.SemaphoreType.DMA for _ in range(NB)],
        ],
    )
    return f(emb, vals_p, row_p, col_p, bnd)


def _dense_body(p_ref, w_ref, all_ref, emb_out_ref, all_out_ref):
    h = jnp.dot(p_ref[...], w_ref[...], preferred_element_type=jnp.float32)
    h = jnp.maximum(h, 0.0)
    nrm = jnp.sqrt(jnp.sum(h * h, axis=1, keepdims=True))
    h = h / jnp.maximum(nrm, 1e-12)
    emb_out_ref[...] = h.astype(jnp.bfloat16)
    all_out_ref[...] = all_ref[...] + h


def _dense(p, W, all_emb):
    N = all_emb.shape[0]
    BLK = 1000
    return pl.pallas_call(
        _dense_body,
        grid=(N // BLK,),
        in_specs=[
            pl.BlockSpec((BLK, D), lambda i: (i, 0)),
            pl.BlockSpec((D, D), lambda i: (0, 0)),
            pl.BlockSpec((BLK, D), lambda i: (i, 0)),
        ],
        out_specs=[
            pl.BlockSpec((BLK, D), lambda i: (i, 0)),
            pl.BlockSpec((BLK, D), lambda i: (i, 0)),
        ],
        out_shape=[
            jax.ShapeDtypeStruct((N, D), jnp.bfloat16),
            jax.ShapeDtypeStruct((N, D), jnp.float32),
        ],
    )(p, W, all_emb)


def kernel(edge_vals, user_table, item_table, W0, W1, W2, edge_row, edge_col):
    n_users = user_table.shape[0]
    N = n_users + item_table.shape[0]
    emb = jnp.concatenate([user_table, item_table], axis=0)
    all_emb = emb
    row_p = jnp.concatenate(
        [edge_row, jnp.full((PAD_E,), N, jnp.int32)])
    col_p = jnp.concatenate([edge_col, jnp.zeros((PAD_E,), jnp.int32)])
    vals_p = jnp.concatenate([edge_vals, jnp.zeros((PAD_E,), jnp.float32)])
    bnd = _count(row_p)
    perm_idx = jnp.array(PERM, jnp.int32)

    def to_i32(emb_bf):
        return lax.bitcast_convert_type(
            emb_bf.reshape(N, D // 2, 2), jnp.int32)

    emb_i = to_i32(emb.astype(jnp.bfloat16))
    for W in (W0, W1, W2):
        p = _spmm(emb_i, vals_p, row_p, col_p, bnd)
        emb_bf, all_emb = _dense(p, W[perm_idx, :], all_emb)
        emb_i = to_i32(emb_bf)
    return all_emb[:n_users], all_emb[n_users:]


# speedup vs baseline: 1.1696x; 1.1696x over previous
import functools

import jax
import jax.numpy as jnp
from jax import lax
from jax.experimental import pallas as pl
from jax.experimental.pallas import tpu as pltpu
from jax.experimental.pallas import tpu_sc as plsc

D = 128
L = 16
CHUNK = 128
NB = 3
NC = 2
NS = 16
NW = NC * NS
RPT = 312
PAD_E = 1024
SLAB = 336
DUMMY = 329


def _count_body(nblk, er_ref, out_ref):
    ids = er_ref[...]
    for w in range(NW + 1):
        bound = RPT * w if w < NW else 10000
        cnt = jnp.sum((ids < bound).astype(jnp.int32))
        out_ref[w] = jnp.full((D,), cnt, jnp.int32)
    for w in range(NW + 1, 40):
        out_ref[w] = jnp.zeros((D,), jnp.int32)


def _count(edge_row_p):
    EP = edge_row_p.shape[0]
    nblk = EP // D
    return pl.pallas_call(
        functools.partial(_count_body, nblk),
        in_specs=[pl.BlockSpec((nblk, D), lambda: (0, 0))],
        out_specs=pl.BlockSpec((40, D), lambda: (0, 0)),
        out_shape=jax.ShapeDtypeStruct((40, D), jnp.int32),
    )(edge_row_p.reshape(nblk, D))


def _spmm_body(N, emb_hbm, e3_hbm, bnd_hbm, out_hbm,
               rows, e3b, bndv, slab, gsem, isem):
    c = lax.axis_index("c")
    s = lax.axis_index("s")
    w = s * NC + c
    rowbase = w * RPT
    nrows = jnp.where(w == NW - 1, RPT + 16, RPT)

    pltpu.sync_copy(bnd_hbm, bndv)
    estart = bndv[w, pl.ds(0, L)][0]
    eend = bndv[w + 1, pl.ds(0, L)][0]
    c0 = lax.div(estart, CHUNK)
    nch = jnp.maximum(lax.div(eend - c0 * CHUNK + CHUNK - 1, CHUNK), 1)
    nloop = lax.div(nch + NB - 1, NB)

    def _zero_row(i, carry):
        for k in range(D // L):
            slab[i, pl.ds(k * L, L)] = jnp.zeros((L,), jnp.float32)
        return carry
    lax.fori_loop(0, SLAB, _zero_row, 0)

    def _start_idx(b, jj):
        pltpu.async_copy(e3_hbm.at[c0 + jj], e3b[b], isem[b])

    def _wait_idx(b, jj):
        pltpu.make_async_copy(e3_hbm.at[c0 + jj], e3b[b], isem[b]).wait()

    def _start_gather(b):
        pltpu.async_copy(emb_hbm.at[e3b[b].at[0]], rows[b], gsem[b])

    def _wait_gather(b):
        pltpu.make_async_copy(emb_hbm.at[e3b[b].at[0]], rows[b],
                              gsem[b]).wait()

    def _process(b):
        def _grp(g, carry):
            acc = [jnp.zeros((L,), jnp.float32) for _ in range(D // L)]
            cur = jnp.int32(DUMMY)
            vgrp = lax.bitcast_convert_type(
                e3b[b][2, pl.ds(g * L, L)], jnp.float32)
            rgrp = e3b[b][1, pl.ds(g * L, L)]
            for e16 in range(L):
                row_e = rgrp[e16]
                sval = vgrp[e16]
                rel = row_e - rowbase
                oor = (rel < 0) | (rel >= nrows)
                rel_c = jnp.where(oor, DUMMY, rel)
                sval = jnp.where(oor, 0.0, sval)
                pred = rel_c != cur
                if e16:
                    @pl.when(pred)
                    def _flush():
                        for k in range(D // L):
                            sl = pl.ds(k * L, L)
                            slab[cur, sl] = slab[cur, sl] + acc[k]
                e = g * L + e16
                keep = jnp.where(pred, 0.0, 1.0)
                for k in range(D // L):
                    fresh = rows[b][e, pl.ds(k * L, L)] * sval
                    acc[k] = fresh + acc[k] * keep
                cur = jnp.where(pred, rel_c, cur)
            for k in range(D // L):
                sl = pl.ds(k * L, L)
                slab[cur, sl] = slab[cur, sl] + acc[k]
            return carry
        lax.fori_loop(0, CHUNK // L, _grp, 0)

    _start_idx(0, 0)
    _start_idx(1, 1)
    _wait_idx(0, 0)
    _start_gather(0)

    def _ring(j, carry):
        for b in range(NB):
            jj = j * NB + b
            _wait_gather(b)
            _start_idx((b + 2) % NB, jj + 2)
            _wait_idx((b + 1) % NB, jj + 1)
            _start_gather((b + 1) % NB)
            _process(b)
        return carry
    lax.fori_loop(0, nloop, _ring, 0)

    _wait_gather(0)
    _wait_idx(1, nloop * NB + 1)

    pltpu.sync_copy(slab.at[pl.ds(0, RPT)], out_hbm.at[pl.ds(rowbase, RPT)])
    @pl.when(w == NW - 1)
    def _write_tail():
        pltpu.sync_copy(slab.at[pl.ds(RPT, 16)],
                        out_hbm.at[pl.ds(rowbase + RPT, 16)])


def _spmm(emb, e3, bnd):
    N = emb.shape[0]
    mesh = plsc.VectorSubcoreMesh(core_axis_name="c", subcore_axis_name="s")
    f = pl.kernel(
        functools.partial(_spmm_body, N),
        out_type=jax.ShapeDtypeStruct((N, D), jnp.float32),
        mesh=mesh,
        scratch_types=[
            [pltpu.VMEM((CHUNK, D), jnp.float32) for _ in range(NB)],
            [pltpu.VMEM((3, CHUNK), jnp.int32) for _ in range(NB)],
            pltpu.VMEM((40, D), jnp.int32),
            pltpu.VMEM((SLAB, D), jnp.float32),
            [pltpu.SemaphoreType.DMA for _ in range(NB)],
            [pltpu.SemaphoreType.DMA for _ in range(NB)],
        ],
    )
    return f(emb, e3, bnd)


def _dense_body(p_ref, w_ref, all_ref, emb_out_ref, all_out_ref):
    h = jnp.dot(p_ref[...], w_ref[...], preferred_element_type=jnp.float32)
    h = jnp.maximum(h, 0.0)
    nrm = jnp.sqrt(jnp.sum(h * h, axis=1, keepdims=True))
    h = h / jnp.maximum(nrm, 1e-12)
    emb_out_ref[...] = h
    all_out_ref[...] = all_ref[...] + h


def _dense(p, W, all_emb):
    N = all_emb.shape[0]
    BLK = 1000
    return pl.pallas_call(
        _dense_body,
        grid=(N // BLK,),
        in_specs=[
            pl.BlockSpec((BLK, D), lambda i: (i, 0)),
            pl.BlockSpec((D, D), lambda i: (0, 0)),
            pl.BlockSpec((BLK, D), lambda i: (i, 0)),
        ],
        out_specs=[
            pl.BlockSpec((BLK, D), lambda i: (i, 0)),
            pl.BlockSpec((BLK, D), lambda i: (i, 0)),
        ],
        out_shape=[
            jax.ShapeDtypeStruct((N, D), jnp.float32),
            jax.ShapeDtypeStruct((N, D), jnp.float32),
        ],
    )(p, W, all_emb)


def kernel(edge_vals, user_table, item_table, W0, W1, W2, edge_row, edge_col):
    n_users = user_table.shape[0]
    N = n_users + item_table.shape[0]
    emb = jnp.concatenate([user_table, item_table], axis=0)
    all_emb = emb
    row_p = jnp.concatenate(
        [edge_row, jnp.full((PAD_E,), N, jnp.int32)])
    col_p = jnp.concatenate([edge_col, jnp.zeros((PAD_E,), jnp.int32)])
    vals_p = jnp.concatenate([edge_vals, jnp.zeros((PAD_E,), jnp.float32)])
    bnd = _count(row_p)
    ncp = row_p.shape[0] // CHUNK
    vals_i = lax.bitcast_convert_type(vals_p, jnp.int32)
    e3 = jnp.stack([col_p.reshape(ncp, CHUNK), row_p.reshape(ncp, CHUNK),
                    vals_i.reshape(ncp, CHUNK)], axis=1)
    for W in (W0, W1, W2):
        p = _spmm(emb, e3, bnd)
        emb, all_emb = _dense(p, W, all_emb)
    return all_emb[:n_users], all_emb[n_users:]
